# Initial kernel scaffold; baseline (speedup 1.0000x reference)
import jax
import jax.numpy as jnp
from jax.experimental import pallas as pl


def kernel(x, W, training):
    raise NotImplementedError("write your pallas kernel here")


# speedup vs baseline: 4.6389x; 4.6389x over previous
import functools

import jax
import jax.numpy as jnp
from jax.experimental import pallas as pl
from jax.experimental.pallas import tpu as pltpu

_TOP_K = 8


def _gate_body(x_ref, w_ref, sparse_ref, topi_ref, loss_ref, acc_ref,
               *, tokens_total, num_experts):
    i = pl.program_id(0)
    logits = jax.lax.dot_general(
        x_ref[...], w_ref[...],
        dimension_numbers=(((1,), (1,)), ((), ())),
        preferred_element_type=jnp.float32)
    m = jnp.max(logits, axis=1, keepdims=True)
    e = jnp.exp(logits - m)
    s = jnp.sum(e, axis=1, keepdims=True)
    w = e / s

    colsum = jnp.sum(w, axis=0, keepdims=True)

    @pl.when(i == 0)
    def _():
        acc_ref[...] = colsum

    @pl.when(i > 0)
    def _():
        acc_ref[...] = acc_ref[...] + colsum

    col = jax.lax.broadcasted_iota(jnp.int32, w.shape, 1)
    wk = w
    selected = jnp.zeros(w.shape, jnp.bool_)
    idx_cols = []
    for _ in range(_TOP_K):
        mx = jnp.max(wk, axis=1, keepdims=True)
        ismax = wk == mx
        idxv = jnp.min(jnp.where(ismax, col, num_experts), axis=1,
                       keepdims=True)
        sel = col == idxv
        selected = jnp.logical_or(selected, sel)
        wk = jnp.where(sel, -1.0, wk)
        idx_cols.append(idxv)

    topi_ref[...] = jnp.concatenate(idx_cols, axis=1)
    picked = jnp.where(selected, w, 0.0)
    tsum = jnp.sum(picked, axis=1, keepdims=True)
    sparse_ref[...] = picked / (tsum + 1e-9)

    @pl.when(i == pl.num_programs(0) - 1)
    def _():
        frac = acc_ref[...] / tokens_total
        target = 1.0 / num_experts
        loss_ref[0, 0] = jnp.sum((frac - target) ** 2) * num_experts


def kernel(x, W, training):
    del training
    tokens, d_model = x.shape
    num_experts = W.shape[0]
    block = 512
    grid = tokens // block

    sparse, topi, loss = pl.pallas_call(
        functools.partial(_gate_body, tokens_total=tokens,
                          num_experts=num_experts),
        grid=(grid,),
        in_specs=[
            pl.BlockSpec((block, d_model), lambda i: (i, 0)),
            pl.BlockSpec((num_experts, d_model), lambda i: (0, 0)),
        ],
        out_specs=[
            pl.BlockSpec((block, num_experts), lambda i: (i, 0)),
            pl.BlockSpec((block, _TOP_K), lambda i: (i, 0)),
            pl.BlockSpec(memory_space=pltpu.SMEM),
        ],
        out_shape=[
            jax.ShapeDtypeStruct((tokens, num_experts), jnp.float32),
            jax.ShapeDtypeStruct((tokens, _TOP_K), jnp.int32),
            jax.ShapeDtypeStruct((1, 1), jnp.float32),
        ],
        scratch_shapes=[pltpu.VMEM((1, num_experts), jnp.float32)],
    )(x, W)
    return (sparse, topi, loss[0, 0])
